# baseline (speedup 1.0000x reference)
import functools
import jax
import jax.numpy as jnp
from jax import lax
from jax.experimental import pallas as pl
from jax.experimental.pallas import tpu as pltpu
from jax.experimental.pallas import tpu_sc as plsc

B, N, D_MODEL = 4, 8192, 1024
EMBED_DIM = 1536
N_REGIONS = 2048

R_BLK = 256
P_BLK = 512
N_CHUNKS = 4
CP = N // N_CHUNKS
NW = 32
B_PER_W = CP // NW
SUB = 4
ROWS_SUB = B_PER_W // SUB


def _cond_kernel(e_ref, w_ref, r_ref, o_ref):
    m = jax.lax.dot_general(
        e_ref[...].astype(jnp.bfloat16), w_ref[...].astype(jnp.bfloat16),
        dimension_numbers=(((1,), (1,)), ((), ())),
        preferred_element_type=jnp.float32,
    ) + r_ref[...]
    o_ref[...] = m.astype(jnp.bfloat16)


def _sc_gather_kernel(table_hbm, idx_hbm, out_hbm, idx_v, rows, gsems, ssems):
    wid = lax.axis_index("s") * 2 + lax.axis_index("c")
    base = wid * B_PER_W
    pltpu.sync_copy(idx_hbm.at[pl.ds(base, B_PER_W)], idx_v)
    gathers = []
    for c in range(SUB):
        gathers.append(pltpu.async_copy(
            table_hbm.at[idx_v.at[pl.ds(c * ROWS_SUB, ROWS_SUB)]],
            rows.at[c], gsems.at[c]))
    stores = []
    for c in range(SUB):
        gathers[c].wait()
        stores.append(pltpu.async_copy(
            rows.at[c], out_hbm.at[pl.ds(base + c * ROWS_SUB, ROWS_SUB)],
            ssems.at[c]))
    for s in stores:
        s.wait()


def _add0_kernel(canvas_ref, cond_ref, out_ref):
    out_ref[...] = canvas_ref[...] + cond_ref[...].astype(jnp.float32)[None]


def _addk_kernel(_acc_ref, canvas_ref, cond_ref, out_ref):
    out_ref[...] = canvas_ref[...] + cond_ref[...].astype(jnp.float32)[None]


def kernel(canvas, region_ids, embeddings, W, residuals):
    table = pl.pallas_call(
        _cond_kernel,
        grid=(N_REGIONS // R_BLK,),
        in_specs=[
            pl.BlockSpec((R_BLK, EMBED_DIM), lambda i: (i, 0)),
            pl.BlockSpec((D_MODEL, EMBED_DIM), lambda i: (0, 0)),
            pl.BlockSpec((R_BLK, D_MODEL), lambda i: (i, 0)),
        ],
        out_specs=pl.BlockSpec((R_BLK, D_MODEL), lambda i: (i, 0)),
        out_shape=jax.ShapeDtypeStruct((N_REGIONS, D_MODEL), jnp.bfloat16),
    )(embeddings, W, residuals)

    table_i32 = jax.lax.bitcast_convert_type(
        table.reshape(N_REGIONS, D_MODEL // 2, 2), jnp.int32)

    ids32 = region_ids.astype(jnp.int32)

    sc_gather = functools.partial(
        pl.kernel,
        mesh=plsc.VectorSubcoreMesh(core_axis_name="c", subcore_axis_name="s"),
        out_type=jax.ShapeDtypeStruct((CP, D_MODEL // 2), jnp.int32),
        scratch_types=[
            pltpu.VMEM((B_PER_W,), jnp.int32),
            pltpu.VMEM((SUB, ROWS_SUB, D_MODEL // 2), jnp.int32),
            pltpu.SemaphoreType.DMA((SUB,)),
            pltpu.SemaphoreType.DMA((SUB,)),
        ],
    )(_sc_gather_kernel)

    cond_chunks = [
        jax.lax.bitcast_convert_type(
            sc_gather(table_i32, lax.dynamic_slice_in_dim(ids32, k * CP, CP)),
            jnp.bfloat16,
        ).reshape(CP, D_MODEL)
        for k in range(N_CHUNKS)
    ]

    blk_per_chunk = CP // P_BLK
    out = None
    for k in range(N_CHUNKS):
        canvas_spec = pl.BlockSpec(
            (1, P_BLK, D_MODEL),
            functools.partial(lambda kk, i, b: (b, i + kk * blk_per_chunk, 0), k),
        )
        cond_spec = pl.BlockSpec((P_BLK, D_MODEL), lambda i, b: (i, 0))
        out_spec = pl.BlockSpec(
            (1, P_BLK, D_MODEL),
            functools.partial(lambda kk, i, b: (b, i + kk * blk_per_chunk, 0), k),
        )
        if k == 0:
            out = pl.pallas_call(
                _add0_kernel,
                grid=(blk_per_chunk, B),
                in_specs=[canvas_spec, cond_spec],
                out_specs=out_spec,
                out_shape=jax.ShapeDtypeStruct((B, N, D_MODEL), jnp.float32),
            )(canvas, cond_chunks[0])
        else:
            out = pl.pallas_call(
                _addk_kernel,
                grid=(blk_per_chunk, B),
                in_specs=[
                    pl.BlockSpec(memory_space=pl.ANY),
                    canvas_spec,
                    cond_spec,
                ],
                out_specs=out_spec,
                out_shape=jax.ShapeDtypeStruct((B, N, D_MODEL), jnp.float32),
                input_output_aliases={0: 0},
            )(out, canvas, cond_chunks[k])

    return out


# speedup vs baseline: 2.0277x; 2.0277x over previous
import functools
import jax
import jax.numpy as jnp
from jax import lax
from jax.experimental import pallas as pl
from jax.experimental.pallas import tpu as pltpu
from jax.experimental.pallas import tpu_sc as plsc

B, N, D_MODEL = 4, 8192, 1024
EMBED_DIM = 1536
N_REGIONS = 2048

R_BLK = 256
P_BLK = 512
N_CHUNKS = 4
CP = N // N_CHUNKS
NW = 32
B_PER_W = CP // NW
SUB = 4
ROWS_SUB = B_PER_W // SUB


def _cond_kernel(e_ref, w_ref, r_ref, o_ref):
    m = jax.lax.dot_general(
        e_ref[...].astype(jnp.bfloat16), w_ref[...].astype(jnp.bfloat16),
        dimension_numbers=(((1,), (1,)), ((), ())),
        preferred_element_type=jnp.float32,
    ) + r_ref[...]
    o_ref[...] = m


def _sc_gather_kernel(table_hbm, idx_hbm, out_hbm, idx_v, rows, gsems, ssems):
    wid = lax.axis_index("s") * 2 + lax.axis_index("c")
    base = wid * B_PER_W
    pltpu.sync_copy(idx_hbm.at[pl.ds(base, B_PER_W)], idx_v)
    gathers = []
    for c in range(SUB):
        gathers.append(pltpu.async_copy(
            table_hbm.at[idx_v.at[pl.ds(c * ROWS_SUB, ROWS_SUB)]],
            rows.at[c], gsems.at[c]))
    stores = []
    for c in range(SUB):
        gathers[c].wait()
        stores.append(pltpu.async_copy(
            rows.at[c], out_hbm.at[pl.ds(base + c * ROWS_SUB, ROWS_SUB)],
            ssems.at[c]))
    for s in stores:
        s.wait()


def _add0_kernel(canvas_ref, cond_ref, out_ref):
    out_ref[...] = canvas_ref[...] + cond_ref[...][None]


def _addk_kernel(_acc_ref, canvas_ref, cond_ref, out_ref):
    out_ref[...] = canvas_ref[...] + cond_ref[...][None]


def kernel(canvas, region_ids, embeddings, W, residuals):
    table = pl.pallas_call(
        _cond_kernel,
        grid=(N_REGIONS // R_BLK,),
        in_specs=[
            pl.BlockSpec((R_BLK, EMBED_DIM), lambda i: (i, 0)),
            pl.BlockSpec((D_MODEL, EMBED_DIM), lambda i: (0, 0)),
            pl.BlockSpec((R_BLK, D_MODEL), lambda i: (i, 0)),
        ],
        out_specs=pl.BlockSpec((R_BLK, D_MODEL), lambda i: (i, 0)),
        out_shape=jax.ShapeDtypeStruct((N_REGIONS, D_MODEL), jnp.float32),
    )(embeddings, W, residuals)

    ids32 = region_ids.astype(jnp.int32)

    sc_gather = functools.partial(
        pl.kernel,
        mesh=plsc.VectorSubcoreMesh(core_axis_name="c", subcore_axis_name="s"),
        out_type=jax.ShapeDtypeStruct((CP, D_MODEL), jnp.float32),
        scratch_types=[
            pltpu.VMEM((B_PER_W,), jnp.int32),
            pltpu.VMEM((SUB, ROWS_SUB, D_MODEL), jnp.float32),
            pltpu.SemaphoreType.DMA((SUB,)),
            pltpu.SemaphoreType.DMA((SUB,)),
        ],
    )(_sc_gather_kernel)

    cond_chunks = [
        sc_gather(table, lax.dynamic_slice_in_dim(ids32, k * CP, CP))
        for k in range(N_CHUNKS)
    ]

    blk_per_chunk = CP // P_BLK
    out = None
    for k in range(N_CHUNKS):
        canvas_spec = pl.BlockSpec(
            (1, P_BLK, D_MODEL),
            functools.partial(lambda kk, i, b: (b, i + kk * blk_per_chunk, 0), k),
        )
        cond_spec = pl.BlockSpec((P_BLK, D_MODEL), lambda i, b: (i, 0))
        out_spec = pl.BlockSpec(
            (1, P_BLK, D_MODEL),
            functools.partial(lambda kk, i, b: (b, i + kk * blk_per_chunk, 0), k),
        )
        if k == 0:
            out = pl.pallas_call(
                _add0_kernel,
                grid=(blk_per_chunk, B),
                in_specs=[canvas_spec, cond_spec],
                out_specs=out_spec,
                out_shape=jax.ShapeDtypeStruct((B, N, D_MODEL), jnp.float32),
            )(canvas, cond_chunks[0])
        else:
            out = pl.pallas_call(
                _addk_kernel,
                grid=(blk_per_chunk, B),
                in_specs=[
                    pl.BlockSpec(memory_space=pl.ANY),
                    canvas_spec,
                    cond_spec,
                ],
                out_specs=out_spec,
                out_shape=jax.ShapeDtypeStruct((B, N, D_MODEL), jnp.float32),
                input_output_aliases={0: 0},
            )(out, canvas, cond_chunks[k])

    return out


# speedup vs baseline: 2.2396x; 1.1045x over previous
import functools
import jax
import jax.numpy as jnp
from jax import lax
from jax.experimental import pallas as pl
from jax.experimental.pallas import tpu as pltpu
from jax.experimental.pallas import tpu_sc as plsc

B, N, D_MODEL = 4, 8192, 1024
EMBED_DIM = 1536
N_REGIONS = 2048

R_BLK = 256
P_BLK = 512
NW = 32
B_PER_W = N // NW
NBUF = 4
ROWS_SUB = 16
SUB = B_PER_W // ROWS_SUB


def _cond_kernel(e_ref, w_ref, r_ref, o_ref):
    o_ref[...] = jax.lax.dot_general(
        e_ref[...].astype(jnp.bfloat16), w_ref[...].astype(jnp.bfloat16),
        dimension_numbers=(((1,), (1,)), ((), ())),
        preferred_element_type=jnp.float32,
    ) + r_ref[...]


def _sc_gather_kernel(table_hbm, idx_hbm, out_hbm, idx_v, rows, gsems, ssems):
    wid = lax.axis_index("s") * 2 + lax.axis_index("c")
    base = wid * B_PER_W
    pltpu.sync_copy(idx_hbm.at[pl.ds(base, B_PER_W)], idx_v)

    gathers = {}
    stores = {}
    for c in range(NBUF):
        gathers[c] = pltpu.async_copy(
            table_hbm.at[idx_v.at[pl.ds(c * ROWS_SUB, ROWS_SUB)]],
            rows.at[c], gsems.at[c])
    for c in range(SUB):
        gathers[c].wait()
        stores[c] = pltpu.async_copy(
            rows.at[c % NBUF],
            out_hbm.at[pl.ds(base + c * ROWS_SUB, ROWS_SUB)],
            ssems.at[c % NBUF])
        nxt = c + NBUF
        if nxt < SUB:
            stores[c].wait()
            gathers[nxt] = pltpu.async_copy(
                table_hbm.at[idx_v.at[pl.ds(nxt * ROWS_SUB, ROWS_SUB)]],
                rows.at[nxt % NBUF], gsems.at[nxt % NBUF])
        else:
            stores[c].wait()


def _add_kernel(canvas_ref, cond_ref, out_ref):
    out_ref[...] = canvas_ref[...] + cond_ref[...][None]


def kernel(canvas, region_ids, embeddings, W, residuals):
    table = pl.pallas_call(
        _cond_kernel,
        grid=(N_REGIONS // R_BLK,),
        in_specs=[
            pl.BlockSpec((R_BLK, EMBED_DIM), lambda i: (i, 0)),
            pl.BlockSpec((D_MODEL, EMBED_DIM), lambda i: (0, 0)),
            pl.BlockSpec((R_BLK, D_MODEL), lambda i: (i, 0)),
        ],
        out_specs=pl.BlockSpec((R_BLK, D_MODEL), lambda i: (i, 0)),
        out_shape=jax.ShapeDtypeStruct((N_REGIONS, D_MODEL), jnp.float32),
    )(embeddings, W, residuals)

    ids32 = region_ids.astype(jnp.int32)

    sc_gather = functools.partial(
        pl.kernel,
        mesh=plsc.VectorSubcoreMesh(core_axis_name="c", subcore_axis_name="s"),
        out_type=jax.ShapeDtypeStruct((N, D_MODEL), jnp.float32),
        scratch_types=[
            pltpu.VMEM((B_PER_W,), jnp.int32),
            pltpu.VMEM((NBUF, ROWS_SUB, D_MODEL), jnp.float32),
            pltpu.SemaphoreType.DMA((NBUF,)),
            pltpu.SemaphoreType.DMA((NBUF,)),
        ],
    )(_sc_gather_kernel)

    cond_per_pos = sc_gather(table, ids32)

    out = pl.pallas_call(
        _add_kernel,
        grid=(N // P_BLK,),
        in_specs=[
            pl.BlockSpec((B, P_BLK, D_MODEL), lambda i: (0, i, 0)),
            pl.BlockSpec((P_BLK, D_MODEL), lambda i: (i, 0)),
        ],
        out_specs=pl.BlockSpec((B, P_BLK, D_MODEL), lambda i: (0, i, 0)),
        out_shape=jax.ShapeDtypeStruct((B, N, D_MODEL), jnp.float32),
    )(canvas, cond_per_pos)

    return out


# speedup vs baseline: 2.2621x; 1.0101x over previous
import functools
import jax
import jax.numpy as jnp
from jax import lax
from jax.experimental import pallas as pl
from jax.experimental.pallas import tpu as pltpu
from jax.experimental.pallas import tpu_sc as plsc

B, N, D_MODEL = 4, 8192, 1024
EMBED_DIM = 1536
N_REGIONS = 2048

R_BLK = 512
P_BLK = 512
NW = 32
B_PER_W = N // NW
NBUF = 3
ROWS_SUB = 32
SUB = B_PER_W // ROWS_SUB


def _cond_kernel(e_ref, w_ref, r_ref, o_ref):
    o_ref[...] = jax.lax.dot_general(
        e_ref[...].astype(jnp.bfloat16), w_ref[...].astype(jnp.bfloat16),
        dimension_numbers=(((1,), (1,)), ((), ())),
        preferred_element_type=jnp.float32,
    ) + r_ref[...]


def _sc_gather_kernel(table_hbm, idx_hbm, out_hbm, idx_v, rows, gsems, ssems):
    wid = lax.axis_index("s") * 2 + lax.axis_index("c")
    base = wid * B_PER_W
    pltpu.sync_copy(idx_hbm.at[pl.ds(base, B_PER_W)], idx_v)

    gathers = {}
    stores = {}
    for c in range(NBUF):
        gathers[c] = pltpu.async_copy(
            table_hbm.at[idx_v.at[pl.ds(c * ROWS_SUB, ROWS_SUB)]],
            rows.at[c], gsems.at[c])
    for c in range(SUB):
        gathers[c].wait()
        stores[c] = pltpu.async_copy(
            rows.at[c % NBUF],
            out_hbm.at[pl.ds(base + c * ROWS_SUB, ROWS_SUB)],
            ssems.at[c % NBUF])
        nxt = c + NBUF
        if nxt < SUB:
            stores[c].wait()
            gathers[nxt] = pltpu.async_copy(
                table_hbm.at[idx_v.at[pl.ds(nxt * ROWS_SUB, ROWS_SUB)]],
                rows.at[nxt % NBUF], gsems.at[nxt % NBUF])
        else:
            stores[c].wait()


def _add_kernel(canvas_ref, cond_ref, out_ref):
    out_ref[...] = canvas_ref[...] + cond_ref[...][None]


def kernel(canvas, region_ids, embeddings, W, residuals):
    table = pl.pallas_call(
        _cond_kernel,
        grid=(N_REGIONS // R_BLK,),
        in_specs=[
            pl.BlockSpec((R_BLK, EMBED_DIM), lambda i: (i, 0)),
            pl.BlockSpec((D_MODEL, EMBED_DIM), lambda i: (0, 0)),
            pl.BlockSpec((R_BLK, D_MODEL), lambda i: (i, 0)),
        ],
        out_specs=pl.BlockSpec((R_BLK, D_MODEL), lambda i: (i, 0)),
        out_shape=jax.ShapeDtypeStruct((N_REGIONS, D_MODEL), jnp.float32),
    )(embeddings, W, residuals)

    ids32 = region_ids.astype(jnp.int32)

    sc_gather = functools.partial(
        pl.kernel,
        mesh=plsc.VectorSubcoreMesh(core_axis_name="c", subcore_axis_name="s"),
        out_type=jax.ShapeDtypeStruct((N, D_MODEL), jnp.float32),
        scratch_types=[
            pltpu.VMEM((B_PER_W,), jnp.int32),
            pltpu.VMEM((NBUF, ROWS_SUB, D_MODEL), jnp.float32),
            pltpu.SemaphoreType.DMA((NBUF,)),
            pltpu.SemaphoreType.DMA((NBUF,)),
        ],
    )(_sc_gather_kernel)

    cond_per_pos = sc_gather(table, ids32)

    out = pl.pallas_call(
        _add_kernel,
        grid=(N // P_BLK,),
        in_specs=[
            pl.BlockSpec((B, P_BLK, D_MODEL), lambda i: (0, i, 0)),
            pl.BlockSpec((P_BLK, D_MODEL), lambda i: (i, 0)),
        ],
        out_specs=pl.BlockSpec((B, P_BLK, D_MODEL), lambda i: (0, i, 0)),
        out_shape=jax.ShapeDtypeStruct((B, N, D_MODEL), jnp.float32),
    )(canvas, cond_per_pos)

    return out


# speedup vs baseline: 2.2727x; 1.0047x over previous
import functools
import jax
import jax.numpy as jnp
from jax import lax
from jax.experimental import pallas as pl
from jax.experimental.pallas import tpu as pltpu
from jax.experimental.pallas import tpu_sc as plsc

B, N, D_MODEL = 4, 8192, 1024
EMBED_DIM = 1536
N_REGIONS = 2048

R_BLK = 512
P_BLK = 2048
NW = 32
B_PER_W = N // NW
NBUF = 3
ROWS_SUB = 32
SUB = B_PER_W // ROWS_SUB


def _cond_kernel(e_ref, w_ref, r_ref, o_ref):
    o_ref[...] = jax.lax.dot_general(
        e_ref[...].astype(jnp.bfloat16), w_ref[...].astype(jnp.bfloat16),
        dimension_numbers=(((1,), (1,)), ((), ())),
        preferred_element_type=jnp.float32,
    ) + r_ref[...]


def _sc_gather_kernel(table_hbm, idx_hbm, out_hbm, idx_v, rows, gsems, ssems):
    wid = lax.axis_index("s") * 2 + lax.axis_index("c")
    base = wid * B_PER_W
    pltpu.sync_copy(idx_hbm.at[pl.ds(base, B_PER_W)], idx_v)

    gathers = {}
    stores = {}
    for c in range(NBUF):
        gathers[c] = pltpu.async_copy(
            table_hbm.at[idx_v.at[pl.ds(c * ROWS_SUB, ROWS_SUB)]],
            rows.at[c], gsems.at[c])
    for c in range(SUB):
        gathers[c].wait()
        stores[c] = pltpu.async_copy(
            rows.at[c % NBUF],
            out_hbm.at[pl.ds(base + c * ROWS_SUB, ROWS_SUB)],
            ssems.at[c % NBUF])
        nxt = c + NBUF
        if nxt < SUB:
            stores[c].wait()
            gathers[nxt] = pltpu.async_copy(
                table_hbm.at[idx_v.at[pl.ds(nxt * ROWS_SUB, ROWS_SUB)]],
                rows.at[nxt % NBUF], gsems.at[nxt % NBUF])
        else:
            stores[c].wait()


def _add_kernel(canvas_ref, cond_ref, out_ref):
    out_ref[...] = canvas_ref[...] + cond_ref[...][None]


def kernel(canvas, region_ids, embeddings, W, residuals):
    table = pl.pallas_call(
        _cond_kernel,
        grid=(N_REGIONS // R_BLK,),
        in_specs=[
            pl.BlockSpec((R_BLK, EMBED_DIM), lambda i: (i, 0)),
            pl.BlockSpec((D_MODEL, EMBED_DIM), lambda i: (0, 0)),
            pl.BlockSpec((R_BLK, D_MODEL), lambda i: (i, 0)),
        ],
        out_specs=pl.BlockSpec((R_BLK, D_MODEL), lambda i: (i, 0)),
        out_shape=jax.ShapeDtypeStruct((N_REGIONS, D_MODEL), jnp.float32),
    )(embeddings, W, residuals)

    ids32 = region_ids.astype(jnp.int32)

    sc_gather = functools.partial(
        pl.kernel,
        mesh=plsc.VectorSubcoreMesh(core_axis_name="c", subcore_axis_name="s"),
        out_type=jax.ShapeDtypeStruct((N, D_MODEL), jnp.float32),
        scratch_types=[
            pltpu.VMEM((B_PER_W,), jnp.int32),
            pltpu.VMEM((NBUF, ROWS_SUB, D_MODEL), jnp.float32),
            pltpu.SemaphoreType.DMA((NBUF,)),
            pltpu.SemaphoreType.DMA((NBUF,)),
        ],
    )(_sc_gather_kernel)

    cond_per_pos = sc_gather(table, ids32)

    out = pl.pallas_call(
        _add_kernel,
        grid=(N // P_BLK, B),
        in_specs=[
            pl.BlockSpec((1, P_BLK, D_MODEL), lambda i, b: (b, i, 0)),
            pl.BlockSpec((P_BLK, D_MODEL), lambda i, b: (i, 0)),
        ],
        out_specs=pl.BlockSpec((1, P_BLK, D_MODEL), lambda i, b: (b, i, 0)),
        out_shape=jax.ShapeDtypeStruct((B, N, D_MODEL), jnp.float32),
    )(canvas, cond_per_pos)

    return out
